# baseline (speedup 1.0000x reference)
import functools

import jax
import jax.numpy as jnp
from jax import lax
from jax.experimental import pallas as pl
from jax.experimental.pallas import tpu as pltpu
from jax.experimental.pallas import tpu_sc as plsc

T = 2048
D = 768
E = 8
HID = 3072
K = 2

BT = 256
BH = 3072
G = (T * K) // BT + E
NH = HID // BH
NSLOT = G * BT

DP = D // 2

_SQRT_HALF = 0.7071067811865476

SC_CORES = 2
SC_SUBCORES = 16
SC_WORKERS = SC_CORES * SC_SUBCORES


def _gating_body(x_ref, gw_ref, gb_ref, i0_ref, i1_ref, p0_ref, p1_ref):
    scores = jnp.dot(x_ref[...], gw_ref[...], preferred_element_type=jnp.float32)
    scores = scores + gb_ref[...]
    iota = lax.broadcasted_iota(jnp.int32, (T, E), 1)
    m0 = jnp.max(scores, axis=1, keepdims=True)
    i0 = jnp.min(jnp.where(scores == m0, iota, E), axis=1, keepdims=True)
    masked = jnp.where(iota == i0, -jnp.inf, scores)
    m1 = jnp.max(masked, axis=1, keepdims=True)
    i1 = jnp.min(jnp.where(masked == m1, iota, E), axis=1, keepdims=True)
    e1 = jnp.exp(m1 - m0)
    p0 = 1.0 / (1.0 + e1)
    i0_ref[...] = i0
    i1_ref[...] = i1
    p0_ref[...] = p0
    p1_ref[...] = e1 * p0


def _gating(x, gate_w, gate_b):
    out_shape = (
        jax.ShapeDtypeStruct((T, 1), jnp.int32),
        jax.ShapeDtypeStruct((T, 1), jnp.int32),
        jax.ShapeDtypeStruct((T, 1), jnp.float32),
        jax.ShapeDtypeStruct((T, 1), jnp.float32),
    )
    return pl.pallas_call(_gating_body, out_shape=out_shape)(
        x, gate_w, gate_b.reshape(1, E)
    )


def _make_sc_gather():
    nw = SC_WORKERS
    bpw = NSLOT // nw
    mesh = plsc.VectorSubcoreMesh(core_axis_name="c", subcore_axis_name="s", num_cores=SC_CORES, num_subcores=SC_SUBCORES)

    @functools.partial(
        pl.kernel,
        out_type=jax.ShapeDtypeStruct((NSLOT, DP), jnp.int32),
        mesh=mesh,
        scratch_types=[
            pltpu.VMEM((bpw,), jnp.int32),
            pltpu.VMEM((bpw, DP), jnp.int32),
            pltpu.SemaphoreType.DMA,
            pltpu.SemaphoreType.DMA,
        ],
    )
    def gather_k(tok_hbm, x_hbm, out_hbm, idx_v, rows_v, isem, rsem):
        wid = lax.axis_index("s") * SC_CORES + lax.axis_index("c")
        base = wid * bpw
        pltpu.async_copy(tok_hbm.at[pl.ds(base, bpw)], idx_v, isem).wait()

        def issue_rows(j, carry):
            vec = idx_v[pl.ds(j * 16, 16)]
            for k in range(16):
                pltpu.async_copy(x_hbm.at[vec[k]], rows_v.at[j * 16 + k], rsem)
            return carry

        lax.fori_loop(0, bpw // 16, issue_rows, 0)
        pltpu.make_async_copy(x_hbm.at[pl.ds(0, bpw)], rows_v, rsem).wait()
        pltpu.sync_copy(rows_v, out_hbm.at[pl.ds(base, bpw)])

    return gather_k


_sc_gather = functools.cache(_make_sc_gather)


def _mlp_body(
    be_ref, nreal_ref, tok_ref, xb_ref, w1_ref, b1_ref, w2_ref, b2_ref, p_ref,
    y_ref, w1s_ref, w2s_ref
):
    g = pl.program_id(0)

    @pl.when(g < nreal_ref[0])
    def _():
        prev = be_ref[jnp.maximum(g - 1, 0)]
        @pl.when(jnp.logical_or(g == 0, be_ref[g] != prev))
        def _():
            w1s_ref[...] = w1_ref[0].astype(jnp.bfloat16)
            w2s_ref[...] = w2_ref[0].astype(jnp.bfloat16)

        tok = tok_ref[0, 0]
        onehot = (tok[:, None] == lax.broadcasted_iota(jnp.int32, (BT, T), 1)
                  ).astype(jnp.bfloat16)
        xg = jnp.dot(onehot, xb_ref[...], preferred_element_type=jnp.float32)
        h = jnp.dot(
            xg.astype(jnp.bfloat16), w1s_ref[...],
            preferred_element_type=jnp.float32,
        )
        h = h + b1_ref[0]
        h = 0.5 * h * (1.0 + lax.erf(h * _SQRT_HALF))
        contrib = jnp.dot(
            h.astype(jnp.bfloat16),
            w2s_ref[...],
            preferred_element_type=jnp.float32,
        )
        y_ref[...] = (contrib + b2_ref[0]) * p_ref[...]


def _grouped_mlp(block_expert, nreal, sorted_tok, xb, w1, b1, w2, b2, sorted_p):
    grid_spec = pltpu.PrefetchScalarGridSpec(
        num_scalar_prefetch=2,
        grid=(G, NH),
        in_specs=[
            pl.BlockSpec((1, 1, BT), lambda g, hb, be, nr: (g, 0, 0)),
            pl.BlockSpec((T, D), lambda g, hb, be, nr: (0, 0)),
            pl.BlockSpec((1, D, BH), lambda g, hb, be, nr: (be[g], 0, hb)),
            pl.BlockSpec((1, 1, BH), lambda g, hb, be, nr: (be[g], 0, hb)),
            pl.BlockSpec((1, BH, D), lambda g, hb, be, nr: (be[g], hb, 0)),
            pl.BlockSpec((1, 1, D), lambda g, hb, be, nr: (be[g], 0, 0)),
            pl.BlockSpec((BT, 1), lambda g, hb, be, nr: (g, 0)),
        ],
        out_specs=pl.BlockSpec((BT, D), lambda g, hb, be, nr: (g, 0)),
        scratch_shapes=[
            pltpu.VMEM((D, HID), jnp.bfloat16),
            pltpu.VMEM((HID, D), jnp.bfloat16),
        ],
    )
    return pl.pallas_call(
        _mlp_body,
        grid_spec=grid_spec,
        out_shape=jax.ShapeDtypeStruct((NSLOT, D), jnp.float32),
    )(
        block_expert,
        nreal,
        sorted_tok.reshape(G, 1, BT),
        xb,
        w1,
        b1.reshape(E, 1, HID),
        w2,
        b2.reshape(E, 1, D),
        sorted_p.reshape(NSLOT, 1),
    )


def _make_sc_combine():
    nw = SC_WORKERS
    tw = T // nw
    mesh = plsc.VectorSubcoreMesh(core_axis_name="c", subcore_axis_name="s", num_cores=SC_CORES, num_subcores=SC_SUBCORES)

    @functools.partial(
        pl.kernel,
        out_type=jax.ShapeDtypeStruct((T, D), jnp.float32),
        mesh=mesh,
        scratch_types=[
            pltpu.VMEM((tw,), jnp.int32),
            pltpu.VMEM((tw,), jnp.int32),
            pltpu.VMEM((tw, D), jnp.float32),
            pltpu.VMEM((tw, D), jnp.float32),
            pltpu.SemaphoreType.DMA,
        ],
    )
    def combine_k(d0_hbm, d1_hbm, ys_hbm, out_hbm, i0_v, i1_v, r0_v, r1_v, sem):
        wid = lax.axis_index("s") * SC_CORES + lax.axis_index("c")
        base = wid * tw
        pltpu.sync_copy(d0_hbm.at[pl.ds(base, tw)], i0_v)
        pltpu.sync_copy(d1_hbm.at[pl.ds(base, tw)], i1_v)
        pltpu.async_copy(ys_hbm.at[i0_v], r0_v, sem).wait()
        pltpu.async_copy(ys_hbm.at[i1_v], r1_v, sem).wait()

        def add_row(r, carry):
            for c in range(D // 16):
                sl = pl.ds(c * 16, 16)
                r0_v[r, sl] = r0_v[r, sl] + r1_v[r, sl]
            return carry

        lax.fori_loop(0, tw, add_row, 0)
        pltpu.sync_copy(r0_v, out_hbm.at[pl.ds(base, tw)])

    return combine_k


_sc_combine = functools.cache(_make_sc_combine)


def kernel(x, gate_w, gate_b, w1, b1, w2, b2):
    i0, i1, p0, p1 = _gating(x, gate_w, gate_b)
    i0, i1 = i0[:, 0], i1[:, 0]
    p0, p1 = p0[:, 0], p1[:, 0]

    eflat = jnp.concatenate([i0, i1])
    pflat = jnp.concatenate([p0, p1])
    ar = jnp.arange(T, dtype=jnp.int32)
    tok = jnp.concatenate([ar, ar])
    onehot = (eflat[:, None] == jnp.arange(E, dtype=jnp.int32)[None, :]).astype(
        jnp.int32
    )
    incl = jnp.cumsum(onehot, axis=0)
    rank = jnp.take_along_axis(incl, eflat[:, None], axis=1)[:, 0] - 1
    counts = incl[-1]
    nblk = (counts + BT - 1) // BT
    endblk = jnp.cumsum(nblk)
    startblk = endblk - nblk
    dest = rank + startblk[eflat] * BT
    sorted_tok = jnp.zeros((NSLOT,), jnp.int32).at[dest].set(
        tok, unique_indices=True
    )
    sorted_p = jnp.zeros((NSLOT,), jnp.float32).at[dest].set(
        pflat, unique_indices=True
    )
    gidx = jnp.arange(G, dtype=jnp.int32)
    block_expert = jnp.minimum(
        jnp.sum((gidx[:, None] >= endblk[None, :]).astype(jnp.int32), axis=1), E - 1
    ).astype(jnp.int32)

    xb = x.astype(jnp.bfloat16)
    nreal = endblk[-1:].astype(jnp.int32)
    ys = _grouped_mlp(
        block_expert, nreal, sorted_tok, xb, w1, b1, w2, b2, sorted_p
    )
    out = _sc_combine()(dest[:T], dest[T:], ys)
    return out


# speedup vs baseline: 1.1374x; 1.1374x over previous
import functools

import jax
import jax.numpy as jnp
from jax import lax
from jax.experimental import pallas as pl
from jax.experimental.pallas import tpu as pltpu
from jax.experimental.pallas import tpu_sc as plsc

T = 2048
D = 768
E = 8
HID = 3072
K = 2

BT = 256
BH = 3072
G = (T * K) // BT + E
NH = HID // BH
NSLOT = G * BT

DP = D // 2

_SQRT_HALF = 0.7071067811865476

SC_CORES = 2
SC_SUBCORES = 16
SC_WORKERS = SC_CORES * SC_SUBCORES


def _gating_body(x_ref, gw_ref, gb_ref, i0_ref, i1_ref, p0_ref, p1_ref):
    scores = jnp.dot(x_ref[...], gw_ref[...], preferred_element_type=jnp.float32)
    scores = scores + gb_ref[...]
    iota = lax.broadcasted_iota(jnp.int32, (T, E), 1)
    m0 = jnp.max(scores, axis=1, keepdims=True)
    i0 = jnp.min(jnp.where(scores == m0, iota, E), axis=1, keepdims=True)
    masked = jnp.where(iota == i0, -jnp.inf, scores)
    m1 = jnp.max(masked, axis=1, keepdims=True)
    i1 = jnp.min(jnp.where(masked == m1, iota, E), axis=1, keepdims=True)
    e1 = jnp.exp(m1 - m0)
    p0 = 1.0 / (1.0 + e1)
    i0_ref[...] = i0
    i1_ref[...] = i1
    p0_ref[...] = p0
    p1_ref[...] = e1 * p0


def _gating(x, gate_w, gate_b):
    out_shape = (
        jax.ShapeDtypeStruct((T, 1), jnp.int32),
        jax.ShapeDtypeStruct((T, 1), jnp.int32),
        jax.ShapeDtypeStruct((T, 1), jnp.float32),
        jax.ShapeDtypeStruct((T, 1), jnp.float32),
    )
    return pl.pallas_call(_gating_body, out_shape=out_shape)(
        x, gate_w, gate_b.reshape(1, E)
    )


def _make_sc_gather():
    nw = SC_WORKERS
    bpw = NSLOT // nw
    mesh = plsc.VectorSubcoreMesh(core_axis_name="c", subcore_axis_name="s", num_cores=SC_CORES, num_subcores=SC_SUBCORES)

    @functools.partial(
        pl.kernel,
        out_type=jax.ShapeDtypeStruct((NSLOT, DP), jnp.int32),
        mesh=mesh,
        scratch_types=[
            pltpu.VMEM((bpw,), jnp.int32),
            pltpu.VMEM((bpw, DP), jnp.int32),
            pltpu.SemaphoreType.DMA,
            pltpu.SemaphoreType.DMA,
        ],
    )
    def gather_k(tok_hbm, x_hbm, out_hbm, idx_v, rows_v, isem, rsem):
        wid = lax.axis_index("s") * SC_CORES + lax.axis_index("c")
        base = wid * bpw
        pltpu.async_copy(tok_hbm.at[pl.ds(base, bpw)], idx_v, isem).wait()

        def issue_rows(j, carry):
            vec = idx_v[pl.ds(j * 16, 16)]
            for k in range(16):
                pltpu.async_copy(x_hbm.at[vec[k]], rows_v.at[j * 16 + k], rsem)
            return carry

        lax.fori_loop(0, bpw // 16, issue_rows, 0)
        pltpu.make_async_copy(x_hbm.at[pl.ds(0, bpw)], rows_v, rsem).wait()
        pltpu.sync_copy(rows_v, out_hbm.at[pl.ds(base, bpw)])

    return gather_k


_sc_gather = functools.cache(_make_sc_gather)


def _mlp_body(be_ref, nreal_ref, tok_ref, xb_ref, w1_ref, b1_ref, w2_ref, b2_ref, p_ref, y_ref):
    del be_ref
    g = pl.program_id(0)

    @pl.when(g < nreal_ref[0])
    def _():
        tok = tok_ref[0, 0]
        onehot = (tok[:, None] == lax.broadcasted_iota(jnp.int32, (BT, T), 1)
                  ).astype(jnp.bfloat16)
        xg = jnp.dot(onehot, xb_ref[...], preferred_element_type=jnp.float32)
        w1b = w1_ref[0].astype(jnp.bfloat16)
        h = jnp.dot(
            xg.astype(jnp.bfloat16), w1b, preferred_element_type=jnp.float32
        )
        h = h + b1_ref[0]
        h = 0.5 * h * (1.0 + lax.erf(h * _SQRT_HALF))
        contrib = jnp.dot(
            h.astype(jnp.bfloat16),
            w2_ref[0].astype(jnp.bfloat16),
            preferred_element_type=jnp.float32,
        )
        y_ref[...] = (contrib + b2_ref[0]) * p_ref[...]


def _grouped_mlp(block_expert, nreal, sorted_tok, xb, w1, b1, w2, b2, sorted_p):
    grid_spec = pltpu.PrefetchScalarGridSpec(
        num_scalar_prefetch=2,
        grid=(G, NH),
        in_specs=[
            pl.BlockSpec((1, 1, BT), lambda g, hb, be, nr: (g, 0, 0)),
            pl.BlockSpec((T, D), lambda g, hb, be, nr: (0, 0)),
            pl.BlockSpec((1, D, BH), lambda g, hb, be, nr: (be[g], 0, hb)),
            pl.BlockSpec((1, 1, BH), lambda g, hb, be, nr: (be[g], 0, hb)),
            pl.BlockSpec((1, BH, D), lambda g, hb, be, nr: (be[g], hb, 0)),
            pl.BlockSpec((1, 1, D), lambda g, hb, be, nr: (be[g], 0, 0)),
            pl.BlockSpec((BT, 1), lambda g, hb, be, nr: (g, 0)),
        ],
        out_specs=pl.BlockSpec((BT, D), lambda g, hb, be, nr: (g, 0)),
    )
    return pl.pallas_call(
        _mlp_body,
        grid_spec=grid_spec,
        out_shape=jax.ShapeDtypeStruct((NSLOT, D), jnp.float32),
    )(
        block_expert,
        nreal,
        sorted_tok.reshape(G, 1, BT),
        xb,
        w1,
        b1.reshape(E, 1, HID),
        w2,
        b2.reshape(E, 1, D),
        sorted_p.reshape(NSLOT, 1),
    )


def _make_sc_combine():
    nw = SC_WORKERS
    tw = T // nw
    mesh = plsc.VectorSubcoreMesh(core_axis_name="c", subcore_axis_name="s", num_cores=SC_CORES, num_subcores=SC_SUBCORES)

    @functools.partial(
        pl.kernel,
        out_type=jax.ShapeDtypeStruct((T, D), jnp.float32),
        mesh=mesh,
        scratch_types=[
            pltpu.VMEM((tw,), jnp.int32),
            pltpu.VMEM((tw,), jnp.int32),
            pltpu.VMEM((tw, D), jnp.float32),
            pltpu.VMEM((tw, D), jnp.float32),
            pltpu.SemaphoreType.DMA,
        ],
    )
    def combine_k(d0_hbm, d1_hbm, ys_hbm, out_hbm, i0_v, i1_v, r0_v, r1_v, sem):
        wid = lax.axis_index("s") * SC_CORES + lax.axis_index("c")
        base = wid * tw
        pltpu.sync_copy(d0_hbm.at[pl.ds(base, tw)], i0_v)
        pltpu.sync_copy(d1_hbm.at[pl.ds(base, tw)], i1_v)
        pltpu.async_copy(ys_hbm.at[i0_v], r0_v, sem).wait()
        pltpu.async_copy(ys_hbm.at[i1_v], r1_v, sem).wait()

        def add_row(r, carry):
            for c in range(D // 16):
                sl = pl.ds(c * 16, 16)
                r0_v[r, sl] = r0_v[r, sl] + r1_v[r, sl]
            return carry

        lax.fori_loop(0, tw, add_row, 0)
        pltpu.sync_copy(r0_v, out_hbm.at[pl.ds(base, tw)])

    return combine_k


_sc_combine = functools.cache(_make_sc_combine)


def kernel(x, gate_w, gate_b, w1, b1, w2, b2):
    i0, i1, p0, p1 = _gating(x, gate_w, gate_b)
    i0, i1 = i0[:, 0], i1[:, 0]
    p0, p1 = p0[:, 0], p1[:, 0]

    eflat = jnp.concatenate([i0, i1])
    pflat = jnp.concatenate([p0, p1])
    ar = jnp.arange(T, dtype=jnp.int32)
    tok = jnp.concatenate([ar, ar])
    onehot = (eflat[:, None] == jnp.arange(E, dtype=jnp.int32)[None, :]).astype(
        jnp.int32
    )
    incl = jnp.cumsum(onehot, axis=0)
    rank = jnp.sum(incl * onehot, axis=1) - 1
    counts = incl[-1]
    nblk = (counts + BT - 1) // BT
    endblk = jnp.cumsum(nblk)
    startblk = endblk - nblk
    dest = rank + startblk[eflat] * BT
    pair = jnp.stack([tok, lax.bitcast_convert_type(pflat, jnp.int32)], axis=-1)
    sorted_pair = jnp.zeros((NSLOT, 2), jnp.int32).at[dest].set(
        pair, unique_indices=True
    )
    sorted_tok = sorted_pair[:, 0]
    sorted_p = lax.bitcast_convert_type(sorted_pair[:, 1], jnp.float32)
    gidx = jnp.arange(G, dtype=jnp.int32)
    block_expert = jnp.minimum(
        jnp.sum((gidx[:, None] >= endblk[None, :]).astype(jnp.int32), axis=1), E - 1
    ).astype(jnp.int32)

    xb = x.astype(jnp.bfloat16)
    nreal = endblk[-1:].astype(jnp.int32)
    ys = _grouped_mlp(
        block_expert, nreal, sorted_tok, xb, w1, b1, w2, b2, sorted_p
    )
    out = _sc_combine()(dest[:T], dest[T:], ys)
    return out
